# initial kernel scaffold (unmeasured)
import jax
import jax.numpy as jnp
from jax import lax
from jax.experimental import pallas as pl
from jax.experimental.pallas import tpu as pltpu

N_DEV = 32
M = 4096
N_OUT = 2048
CHUNK = M // N_DEV
N_STEPS = 2 * (N_DEV - 1)


def kernel(x, w_mat):
    def body(x_ref, w_ref, out_ref, send_ref, recv_ref, send_sems, recv_sems,
             credit_sem, exit_sem):
        me = lax.axis_index("i")
        left = lax.rem(me - 1 + N_DEV, N_DEV)
        right = lax.rem(me + 1, N_DEV)

        def row(c):
            return pl.ds(c * CHUNK, CHUNK)

        def chunk_idx(s):
            return lax.rem(me - s + 3 * N_DEV, N_DEV)

        barrier_sem = pltpu.get_barrier_semaphore()
        for nbr in (left, right):
            pl.semaphore_signal(barrier_sem, inc=1, device_id=(nbr,),
                                device_id_type=pl.DeviceIdType.MESH)
        pl.semaphore_wait(barrier_sem, 2)

        out_ref[:, :] = jnp.dot(x_ref[:, :], w_ref[:, :],
                                preferred_element_type=jnp.float32)

        for s in range(N_STEPS):
            slot = s % 2
            prev = 1 - slot
            if s == 0:
                send_ref[0, :, :] = out_ref[row(me), :]
            else:
                c = chunk_idx(s)
                rv = recv_ref[prev, :, :]
                if s <= N_DEV - 2:
                    send_ref[slot, :, :] = rv + out_ref[row(c), :]
                elif s == N_DEV - 1:
                    out_ref[row(c), :] = rv + out_ref[row(c), :]
                    send_ref[slot, :, :] = out_ref[row(c), :]
                else:
                    out_ref[row(c), :] = rv
                    send_ref[slot, :, :] = rv
                if s <= N_STEPS - 2:
                    pl.semaphore_signal(credit_sem, inc=1, device_id=(left,),
                                        device_id_type=pl.DeviceIdType.MESH)
            if s >= 2:
                pl.semaphore_wait(credit_sem, 1)
            rdma = pltpu.make_async_remote_copy(
                src_ref=send_ref.at[slot],
                dst_ref=recv_ref.at[slot],
                send_sem=send_sems.at[slot],
                recv_sem=recv_sems.at[slot],
                device_id=(right,),
                device_id_type=pl.DeviceIdType.MESH,
            )
            rdma.start()
            rdma.wait()

        out_ref[row(chunk_idx(N_STEPS)), :] = recv_ref[1, :, :]

        for nbr in (left, right):
            pl.semaphore_signal(exit_sem, inc=1, device_id=(nbr,),
                                device_id_type=pl.DeviceIdType.MESH)
        pl.semaphore_wait(exit_sem, 2)

        y = jnp.maximum(out_ref[:, :], 0.0)
        amax = jnp.max(y)
        scale = amax / 448.0
        v = jnp.minimum(y / scale, 448.0)
        u = lax.bitcast_convert_type(v, jnp.int32)
        u = (u + 0x7FFFF + ((u >> 20) & 1)) & ~0xFFFFF
        q = lax.bitcast_convert_type(u, jnp.float32)
        out_ref[:, :] = q * scale

    return pl.pallas_call(
        body,
        out_shape=jax.ShapeDtypeStruct((M, N_OUT), jnp.float32),
        in_specs=[
            pl.BlockSpec(memory_space=pltpu.VMEM),
            pl.BlockSpec(memory_space=pltpu.VMEM),
        ],
        out_specs=pl.BlockSpec(memory_space=pltpu.VMEM),
        scratch_shapes=[
            pltpu.VMEM((2, CHUNK, N_OUT), jnp.float32),
            pltpu.VMEM((2, CHUNK, N_OUT), jnp.float32),
            pltpu.SemaphoreType.DMA((2,)),
            pltpu.SemaphoreType.DMA((2,)),
            pltpu.SemaphoreType.REGULAR,
            pltpu.SemaphoreType.REGULAR,
        ],
        compiler_params=pltpu.CompilerParams(collective_id=0),
    )(x, w_mat)


# baseline (device time: 907811 ns/iter reference)
import jax
import jax.numpy as jnp
from jax import lax
from jax.experimental import pallas as pl
from jax.experimental.pallas import tpu as pltpu

N_DEV = 32
M = 4096
N_OUT = 2048
CHUNK = M // N_DEV
N_STEPS = 2 * (N_DEV - 1)

import os
USE_CREDITS = os.environ.get("K_NO_CREDITS") != "1"
USE_EXIT_BARRIER = os.environ.get("K_NO_EXITBAR") != "1"
RAW_OUTPUT = os.environ.get("K_RAW") == "1"


def kernel(x, w_mat):
    def body(x_ref, w_ref, out_ref, send_ref, recv_ref, send_sems, recv_sems,
             credit_sem):
        me = lax.axis_index("i")
        left = lax.rem(me - 1 + N_DEV, N_DEV)
        right = lax.rem(me + 1, N_DEV)

        def row(c):
            return pl.ds(c * CHUNK, CHUNK)

        def chunk_idx(s):
            return lax.rem(me - s + 3 * N_DEV, N_DEV)

        barrier_sem = pltpu.get_barrier_semaphore()
        for nbr in (left, right):
            pl.semaphore_signal(barrier_sem, inc=1, device_id=(nbr,),
                                device_id_type=pl.DeviceIdType.MESH)
        pl.semaphore_wait(barrier_sem, 2)

        for c in range(N_DEV):
            sl = pl.ds(c * CHUNK, CHUNK)
            out_ref[sl, :] = jnp.dot(x_ref[sl, :], w_ref[:, :],
                                     preferred_element_type=jnp.float32,
                                     precision=lax.Precision.HIGHEST)

        for s in range(N_STEPS):
            slot = s % 2
            prev = 1 - slot
            if s == 0:
                send_ref[0, :, :] = out_ref[row(me), :]
            else:
                c = chunk_idx(s)
                rv = recv_ref[prev, :, :]
                if s <= N_DEV - 2:
                    send_ref[slot, :, :] = rv + out_ref[row(c), :]
                elif s == N_DEV - 1:
                    out_ref[row(c), :] = rv + out_ref[row(c), :]
                    send_ref[slot, :, :] = out_ref[row(c), :]
                else:
                    out_ref[row(c), :] = rv
                    send_ref[slot, :, :] = rv
                if USE_CREDITS and s <= N_STEPS - 2:
                    pl.semaphore_signal(credit_sem, inc=1, device_id=(left,),
                                        device_id_type=pl.DeviceIdType.MESH)
            if USE_CREDITS and s >= 2:
                pl.semaphore_wait(credit_sem, 1)
            rdma = pltpu.make_async_remote_copy(
                src_ref=send_ref.at[slot],
                dst_ref=recv_ref.at[slot],
                send_sem=send_sems.at[slot],
                recv_sem=recv_sems.at[slot],
                device_id=(right,),
                device_id_type=pl.DeviceIdType.MESH,
            )
            rdma.start()
            rdma.wait()

        out_ref[row(chunk_idx(N_STEPS)), :] = recv_ref[1, :, :]

        if USE_EXIT_BARRIER:
            for nbr in (left, right):
                pl.semaphore_signal(barrier_sem, inc=1, device_id=(nbr,),
                                    device_id_type=pl.DeviceIdType.MESH)
            pl.semaphore_wait(barrier_sem, 2)

        if RAW_OUTPUT:
            return
        maxes = []
        for c in range(N_DEV):
            sl = pl.ds(c * CHUNK, CHUNK)
            y = jnp.maximum(out_ref[sl, :], 0.0)
            out_ref[sl, :] = y
            maxes.append(jnp.max(y))
        amax = jnp.max(jnp.stack(maxes))
        scale = amax / 448.0
        inv_scale = 448.0 / amax
        for c in range(N_DEV):
            sl = pl.ds(c * CHUNK, CHUNK)
            v = jnp.minimum(out_ref[sl, :] * inv_scale, 448.0)
            u = lax.bitcast_convert_type(v, jnp.int32)
            u = (u + 0x7FFFF + ((u >> 20) & 1)) & ~0xFFFFF
            q = lax.bitcast_convert_type(u, jnp.float32)
            out_ref[sl, :] = q * scale

    return pl.pallas_call(
        body,
        out_shape=jax.ShapeDtypeStruct((M, N_OUT), jnp.float32),
        in_specs=[
            pl.BlockSpec(memory_space=pltpu.VMEM),
            pl.BlockSpec(memory_space=pltpu.VMEM),
        ],
        out_specs=pl.BlockSpec(memory_space=pltpu.VMEM),
        scratch_shapes=[
            pltpu.VMEM((2, CHUNK, N_OUT), jnp.float32),
            pltpu.VMEM((2, CHUNK, N_OUT), jnp.float32),
            pltpu.SemaphoreType.DMA((2,)),
            pltpu.SemaphoreType.DMA((2,)),
            pltpu.SemaphoreType.REGULAR,
        ],
        compiler_params=pltpu.CompilerParams(
            collective_id=0,
            vmem_limit_bytes=56 * 1024 * 1024,
        ),
    )(x, w_mat)


# device time: 906034 ns/iter; 1.0020x vs baseline; 1.0020x over previous
import os

import jax
import jax.numpy as jnp
from jax import lax
from jax.experimental import pallas as pl
from jax.experimental.pallas import tpu as pltpu

N_DEV = 32
M = 4096
N_OUT = 2048
HALF = N_OUT // 2
CHUNK = M // N_DEV
N_STEPS = 2 * (N_DEV - 1)

RAW_OUTPUT = os.environ.get("K_RAW") == "1"


def kernel(x, w_mat):
    def body(x_ref, w_ref, out_ref,
             send_r, recv_r, send_l, recv_l,
             ssem_r, rsem_r, ssem_l, rsem_l, credit_sems):
        me = lax.axis_index("i")
        left = lax.rem(me - 1 + N_DEV, N_DEV)
        right = lax.rem(me + 1, N_DEV)

        def row(c):
            return pl.ds(c * CHUNK, CHUNK)

        dirs = (
            (-1, right, left, 0, send_r, recv_r, ssem_r, rsem_r),
            (+1, left, right, HALF, send_l, recv_l, ssem_l, rsem_l),
        )

        def chunk_idx(sgn, s):
            return lax.rem(me + sgn * s + 3 * N_DEV, N_DEV)

        barrier_sem = pltpu.get_barrier_semaphore()
        for nbr in (left, right):
            pl.semaphore_signal(barrier_sem, inc=1, device_id=(nbr,),
                                device_id_type=pl.DeviceIdType.MESH)
        pl.semaphore_wait(barrier_sem, 2)

        for c in range(N_DEV):
            sl = pl.ds(c * CHUNK, CHUNK)
            out_ref[sl, :] = jnp.dot(x_ref[sl, :], w_ref[:, :],
                                     preferred_element_type=jnp.float32,
                                     precision=lax.Precision.HIGHEST)

        maxes = []

        for s in range(N_STEPS):
            slot = s % 2
            prev = 1 - slot
            rdmas = []
            for d, (sgn, to, cred_to, coff, sbuf, rbuf, ssem, rsem) \
                    in enumerate(dirs):
                cols = pl.ds(coff, HALF)
                if s == 0:
                    sbuf[0, :, :] = out_ref[row(me), cols]
                else:
                    c = chunk_idx(sgn, s)
                    rv = rbuf[prev, :, :]
                    if s <= N_DEV - 2:
                        sbuf[slot, :, :] = rv + out_ref[row(c), cols]
                    elif s == N_DEV - 1:
                        v = rv + out_ref[row(c), cols]
                        sbuf[slot, :, :] = v
                        v = jnp.maximum(v, 0.0)
                        out_ref[row(c), cols] = v
                        maxes.append(jnp.max(v))
                    else:
                        sbuf[slot, :, :] = rv
                        v = jnp.maximum(rv, 0.0)
                        out_ref[row(chunk_idx(sgn, s)), cols] = v
                        maxes.append(jnp.max(v))
                    if s <= N_STEPS - 2:
                        pl.semaphore_signal(
                            credit_sems.at[d], inc=1, device_id=(cred_to,),
                            device_id_type=pl.DeviceIdType.MESH)
                if s >= 2:
                    pl.semaphore_wait(credit_sems.at[d], 1)
                rdmas.append(pltpu.make_async_remote_copy(
                    src_ref=sbuf.at[slot],
                    dst_ref=rbuf.at[slot],
                    send_sem=ssem.at[slot],
                    recv_sem=rsem.at[slot],
                    device_id=(to,),
                    device_id_type=pl.DeviceIdType.MESH,
                ))
            for rdma in rdmas:
                rdma.start()
            for rdma in rdmas:
                rdma.wait()

        for sgn, _, _, coff, _, rbuf, _, _ in dirs:
            v = jnp.maximum(rbuf[1, :, :], 0.0)
            out_ref[row(chunk_idx(sgn, N_STEPS)), pl.ds(coff, HALF)] = v
            maxes.append(jnp.max(v))

        for nbr in (left, right):
            pl.semaphore_signal(barrier_sem, inc=1, device_id=(nbr,),
                                device_id_type=pl.DeviceIdType.MESH)
        pl.semaphore_wait(barrier_sem, 2)

        if RAW_OUTPUT:
            return

        amax = jnp.max(jnp.stack(maxes))
        scale = amax / 448.0
        inv_scale = 448.0 / amax
        for c in range(N_DEV):
            sl = pl.ds(c * CHUNK, CHUNK)
            v = jnp.minimum(out_ref[sl, :] * inv_scale, 448.0)
            u = lax.bitcast_convert_type(v, jnp.int32)
            u = (u + 0x7FFFF + ((u >> 20) & 1)) & ~0xFFFFF
            q = lax.bitcast_convert_type(u, jnp.float32)
            out_ref[sl, :] = q * scale

    return pl.pallas_call(
        body,
        out_shape=jax.ShapeDtypeStruct((M, N_OUT), jnp.float32),
        in_specs=[
            pl.BlockSpec(memory_space=pltpu.VMEM),
            pl.BlockSpec(memory_space=pltpu.VMEM),
        ],
        out_specs=pl.BlockSpec(memory_space=pltpu.VMEM),
        scratch_shapes=[
            pltpu.VMEM((2, CHUNK, HALF), jnp.float32),
            pltpu.VMEM((2, CHUNK, HALF), jnp.float32),
            pltpu.VMEM((2, CHUNK, HALF), jnp.float32),
            pltpu.VMEM((2, CHUNK, HALF), jnp.float32),
            pltpu.SemaphoreType.DMA((2,)),
            pltpu.SemaphoreType.DMA((2,)),
            pltpu.SemaphoreType.DMA((2,)),
            pltpu.SemaphoreType.DMA((2,)),
            pltpu.SemaphoreType.REGULAR((2,)),
        ],
        compiler_params=pltpu.CompilerParams(
            collective_id=0,
            vmem_limit_bytes=56 * 1024 * 1024,
        ),
    )(x, w_mat)
